# baseline (speedup 1.0000x reference)
import functools

import jax
import jax.numpy as jnp
from jax.experimental import pallas as pl
from jax.experimental.pallas import tpu as pltpu

_TEMPERATURE = 0.1
_SIGMA_STATIC = 5
_ROW_BLOCK = 64
_LANES = 128


def _cts_loss_kernel(f_ref, acc_ref, fn_ref, *, blk):
    i = pl.program_id(0)
    B = f_ref.shape[0]
    nsel = _SIGMA_STATIC + 1
    inv_t = 1.0 / _TEMPERATURE

    @pl.when(i == 0)
    def _normalize():
        f = f_ref[...]
        norm = jnp.sqrt(jnp.sum(f * f, axis=1, keepdims=True))
        fn_ref[...] = f / jnp.maximum(norm, 1e-12)

    fn = fn_ref[...]
    fblk = fn_ref[pl.ds(i * blk, blk), :]

    sim = jax.lax.dot_general(
        fblk, fn,
        dimension_numbers=(((1,), (1,)), ((), ())),
        preferred_element_type=jnp.float32,
    )

    tops = [jnp.full((blk, _LANES), -3.0, jnp.float32) for _ in range(nsel)]
    es = jnp.zeros((blk, _LANES), jnp.float32)
    for g in range(B // _LANES):
        v = jax.lax.slice(sim, (0, g * _LANES), (blk, (g + 1) * _LANES))
        es = es + jnp.exp(v * inv_t)
        for j in range(nsel):
            hi = jnp.maximum(tops[j], v)
            v = jnp.minimum(tops[j], v)
            tops[j] = hi
    esum = jnp.sum(es, axis=1, keepdims=True)

    work = jnp.concatenate(tops, axis=1)
    top_sum = jnp.zeros((blk, 1), jnp.float32)
    exp_top = jnp.zeros((blk, 1), jnp.float32)
    taken = jnp.zeros((blk, 1), jnp.float32)
    for k in range(nsel):
        m = jnp.max(work, axis=1, keepdims=True)
        eq = work == m
        cnt = jnp.sum(eq.astype(jnp.float32), axis=1, keepdims=True)
        use = jnp.minimum(cnt, nsel - taken)
        contrib = jnp.maximum(use - jnp.maximum(1.0 - taken, 0.0), 0.0)
        top_sum = top_sum + m * contrib
        exp_top = exp_top + jnp.exp(m * inv_t) * use
        taken = taken + use
        if k + 1 < nsel:
            work = jnp.where(eq, -3.0, work)

    neg_sum = esum - exp_top
    row_loss = top_sum * inv_t - _SIGMA_STATIC * jnp.log(neg_sum)
    partial = jnp.sum(row_loss).reshape(1, 1)

    @pl.when(i == 0)
    def _init():
        acc_ref[...] = jnp.zeros((1, 1), jnp.float32)

    acc_ref[...] += partial


def kernel(features, sigma):
    B, D = features.shape
    blk = min(_ROW_BLOCK, B)
    grid = (B // blk,)

    acc = pl.pallas_call(
        functools.partial(_cts_loss_kernel, blk=blk),
        grid=grid,
        in_specs=[pl.BlockSpec((B, D), lambda i: (0, 0))],
        out_specs=pl.BlockSpec((1, 1), lambda i: (0, 0)),
        out_shape=jax.ShapeDtypeStruct((1, 1), jnp.float32),
        scratch_shapes=[pltpu.VMEM((B, D), jnp.float32)],
    )(features)

    loss = -(acc[0, 0] / sigma) / B
    return jnp.maximum(loss, jnp.asarray(0.0, dtype=loss.dtype))


# speedup vs baseline: 1.3879x; 1.3879x over previous
import functools

import jax
import jax.numpy as jnp
from jax.experimental import pallas as pl
from jax.experimental.pallas import tpu as pltpu

_TEMPERATURE = 0.1
_SIGMA_STATIC = 5
_ROW_BLOCK = 128
_LANES = 128


def _cts_loss_kernel(f_ref, acc_ref, fn_ref, *, blk):
    i = pl.program_id(0)
    B = f_ref.shape[0]
    nsel = _SIGMA_STATIC + 1
    inv_t = 1.0 / _TEMPERATURE

    @pl.when(i == 0)
    def _normalize():
        f = f_ref[...]
        norm = jnp.sqrt(jnp.sum(f * f, axis=1, keepdims=True))
        fn_ref[...] = f / jnp.maximum(norm, 1e-12)

    fn = fn_ref[...]
    fblk = fn_ref[pl.ds(i * blk, blk), :]

    sim = jax.lax.dot_general(
        fblk, fn,
        dimension_numbers=(((1,), (1,)), ((), ())),
        preferred_element_type=jnp.float32,
    )

    tops = [jnp.full((blk, _LANES), -3.0, jnp.float32) for _ in range(nsel)]
    es = jnp.zeros((blk, _LANES), jnp.float32)
    for g in range(B // _LANES):
        v = jax.lax.slice(sim, (0, g * _LANES), (blk, (g + 1) * _LANES))
        es = es + jnp.exp(v * inv_t)
        for j in range(nsel):
            hi = jnp.maximum(tops[j], v)
            v = jnp.minimum(tops[j], v)
            tops[j] = hi
    esum = jnp.sum(es, axis=1, keepdims=True)

    work = jnp.concatenate(tops, axis=1)
    top_sum = jnp.zeros((blk, 1), jnp.float32)
    exp_top = jnp.zeros((blk, 1), jnp.float32)
    taken = jnp.zeros((blk, 1), jnp.float32)
    for k in range(nsel):
        m = jnp.max(work, axis=1, keepdims=True)
        eq = work == m
        cnt = jnp.sum(eq.astype(jnp.float32), axis=1, keepdims=True)
        use = jnp.minimum(cnt, nsel - taken)
        contrib = jnp.maximum(use - jnp.maximum(1.0 - taken, 0.0), 0.0)
        top_sum = top_sum + m * contrib
        exp_top = exp_top + jnp.exp(m * inv_t) * use
        taken = taken + use
        if k + 1 < nsel:
            work = jnp.where(eq, -3.0, work)

    neg_sum = esum - exp_top
    row_loss = top_sum * inv_t - _SIGMA_STATIC * jnp.log(neg_sum)
    partial = jnp.sum(row_loss).reshape(1, 1)

    @pl.when(i == 0)
    def _init():
        acc_ref[...] = jnp.zeros((1, 1), jnp.float32)

    acc_ref[...] += partial


def kernel(features, sigma):
    B, D = features.shape
    blk = min(_ROW_BLOCK, B)
    grid = (B // blk,)

    acc = pl.pallas_call(
        functools.partial(_cts_loss_kernel, blk=blk),
        grid=grid,
        in_specs=[pl.BlockSpec((B, D), lambda i: (0, 0))],
        out_specs=pl.BlockSpec((1, 1), lambda i: (0, 0)),
        out_shape=jax.ShapeDtypeStruct((1, 1), jnp.float32),
        scratch_shapes=[pltpu.VMEM((B, D), jnp.float32)],
    )(features)

    loss = -(acc[0, 0] / sigma) / B
    return jnp.maximum(loss, jnp.asarray(0.0, dtype=loss.dtype))


# speedup vs baseline: 1.5140x; 1.0908x over previous
import functools

import jax
import jax.numpy as jnp
from jax.experimental import pallas as pl
from jax.experimental.pallas import tpu as pltpu

_TEMPERATURE = 0.1
_SIGMA_STATIC = 5
_ROW_BLOCK = 256
_LANES = 128


def _cts_loss_kernel(f_ref, acc_ref, fn_ref, *, blk):
    i = pl.program_id(0)
    B = f_ref.shape[0]
    nsel = _SIGMA_STATIC + 1
    inv_t = 1.0 / _TEMPERATURE

    @pl.when(i == 0)
    def _normalize():
        f = f_ref[...]
        norm = jnp.sqrt(jnp.sum(f * f, axis=1, keepdims=True))
        fn_ref[...] = f / jnp.maximum(norm, 1e-12)

    fn = fn_ref[...]
    fblk = fn_ref[pl.ds(i * blk, blk), :]

    sim = jax.lax.dot_general(
        fblk, fn,
        dimension_numbers=(((1,), (1,)), ((), ())),
        preferred_element_type=jnp.float32,
    )

    tops = [jnp.full((blk, _LANES), -3.0, jnp.float32) for _ in range(nsel)]
    es = jnp.zeros((blk, _LANES), jnp.float32)
    for g in range(B // _LANES):
        v = jax.lax.slice(sim, (0, g * _LANES), (blk, (g + 1) * _LANES))
        es = es + jnp.exp(v * inv_t)
        for j in range(nsel):
            hi = jnp.maximum(tops[j], v)
            v = jnp.minimum(tops[j], v)
            tops[j] = hi
    esum = jnp.sum(es, axis=1, keepdims=True)

    work = jnp.concatenate(tops, axis=1)
    top_sum = jnp.zeros((blk, 1), jnp.float32)
    exp_top = jnp.zeros((blk, 1), jnp.float32)
    taken = jnp.zeros((blk, 1), jnp.float32)
    for k in range(nsel):
        m = jnp.max(work, axis=1, keepdims=True)
        eq = work == m
        cnt = jnp.sum(eq.astype(jnp.float32), axis=1, keepdims=True)
        use = jnp.minimum(cnt, nsel - taken)
        contrib = jnp.maximum(use - jnp.maximum(1.0 - taken, 0.0), 0.0)
        top_sum = top_sum + m * contrib
        exp_top = exp_top + jnp.exp(m * inv_t) * use
        taken = taken + use
        if k + 1 < nsel:
            work = jnp.where(eq, -3.0, work)

    neg_sum = esum - exp_top
    row_loss = top_sum * inv_t - _SIGMA_STATIC * jnp.log(neg_sum)
    partial = jnp.sum(row_loss).reshape(1, 1)

    @pl.when(i == 0)
    def _init():
        acc_ref[...] = jnp.zeros((1, 1), jnp.float32)

    acc_ref[...] += partial


def kernel(features, sigma):
    B, D = features.shape
    blk = min(_ROW_BLOCK, B)
    grid = (B // blk,)

    acc = pl.pallas_call(
        functools.partial(_cts_loss_kernel, blk=blk),
        grid=grid,
        in_specs=[pl.BlockSpec((B, D), lambda i: (0, 0))],
        out_specs=pl.BlockSpec((1, 1), lambda i: (0, 0)),
        out_shape=jax.ShapeDtypeStruct((1, 1), jnp.float32),
        scratch_shapes=[pltpu.VMEM((B, D), jnp.float32)],
    )(features)

    loss = -(acc[0, 0] / sigma) / B
    return jnp.maximum(loss, jnp.asarray(0.0, dtype=loss.dtype))


# speedup vs baseline: 1.6150x; 1.0667x over previous
import functools

import jax
import jax.numpy as jnp
from jax.experimental import pallas as pl
from jax.experimental.pallas import tpu as pltpu

_TEMPERATURE = 0.1
_SIGMA_STATIC = 5
_ROW_BLOCK = 512
_LANES = 128


def _cts_loss_kernel(f_ref, acc_ref, fn_ref, *, blk):
    i = pl.program_id(0)
    B = f_ref.shape[0]
    nsel = _SIGMA_STATIC + 1
    inv_t = 1.0 / _TEMPERATURE

    @pl.when(i == 0)
    def _normalize():
        f = f_ref[...]
        norm = jnp.sqrt(jnp.sum(f * f, axis=1, keepdims=True))
        fn_ref[...] = f / jnp.maximum(norm, 1e-12)

    fn = fn_ref[...]
    fblk = fn_ref[pl.ds(i * blk, blk), :]

    sim = jax.lax.dot_general(
        fblk, fn,
        dimension_numbers=(((1,), (1,)), ((), ())),
        preferred_element_type=jnp.float32,
    )

    tops = [jnp.full((blk, _LANES), -3.0, jnp.float32) for _ in range(nsel)]
    es = jnp.zeros((blk, _LANES), jnp.float32)
    for g in range(B // _LANES):
        v = jax.lax.slice(sim, (0, g * _LANES), (blk, (g + 1) * _LANES))
        es = es + jnp.exp(v * inv_t)
        for j in range(nsel):
            hi = jnp.maximum(tops[j], v)
            v = jnp.minimum(tops[j], v)
            tops[j] = hi
    esum = jnp.sum(es, axis=1, keepdims=True)

    work = jnp.concatenate(tops, axis=1)
    top_sum = jnp.zeros((blk, 1), jnp.float32)
    exp_top = jnp.zeros((blk, 1), jnp.float32)
    taken = jnp.zeros((blk, 1), jnp.float32)
    for k in range(nsel):
        m = jnp.max(work, axis=1, keepdims=True)
        eq = work == m
        cnt = jnp.sum(eq.astype(jnp.float32), axis=1, keepdims=True)
        use = jnp.minimum(cnt, nsel - taken)
        contrib = jnp.maximum(use - jnp.maximum(1.0 - taken, 0.0), 0.0)
        top_sum = top_sum + m * contrib
        exp_top = exp_top + jnp.exp(m * inv_t) * use
        taken = taken + use
        if k + 1 < nsel:
            work = jnp.where(eq, -3.0, work)

    neg_sum = esum - exp_top
    row_loss = top_sum * inv_t - _SIGMA_STATIC * jnp.log(neg_sum)
    partial = jnp.sum(row_loss).reshape(1, 1)

    @pl.when(i == 0)
    def _init():
        acc_ref[...] = jnp.zeros((1, 1), jnp.float32)

    acc_ref[...] += partial


def kernel(features, sigma):
    B, D = features.shape
    blk = min(_ROW_BLOCK, B)
    grid = (B // blk,)

    acc = pl.pallas_call(
        functools.partial(_cts_loss_kernel, blk=blk),
        grid=grid,
        in_specs=[pl.BlockSpec((B, D), lambda i: (0, 0))],
        out_specs=pl.BlockSpec((1, 1), lambda i: (0, 0)),
        out_shape=jax.ShapeDtypeStruct((1, 1), jnp.float32),
        scratch_shapes=[pltpu.VMEM((B, D), jnp.float32)],
    )(features)

    loss = -(acc[0, 0] / sigma) / B
    return jnp.maximum(loss, jnp.asarray(0.0, dtype=loss.dtype))


# speedup vs baseline: 1.7085x; 1.0579x over previous
import functools

import jax
import jax.numpy as jnp
from jax.experimental import pallas as pl
from jax.experimental.pallas import tpu as pltpu

_TEMPERATURE = 0.1
_SIGMA_STATIC = 5
_ROW_BLOCK = 1024
_LANES = 128


def _cts_loss_kernel(f_ref, acc_ref, fn_ref, *, blk):
    i = pl.program_id(0)
    B = f_ref.shape[0]
    nsel = _SIGMA_STATIC + 1
    inv_t = 1.0 / _TEMPERATURE

    @pl.when(i == 0)
    def _normalize():
        f = f_ref[...]
        norm = jnp.sqrt(jnp.sum(f * f, axis=1, keepdims=True))
        fn_ref[...] = f / jnp.maximum(norm, 1e-12)

    fn = fn_ref[...]
    fblk = fn_ref[pl.ds(i * blk, blk), :]

    sim = jax.lax.dot_general(
        fblk, fn,
        dimension_numbers=(((1,), (1,)), ((), ())),
        preferred_element_type=jnp.float32,
    )

    tops = [jnp.full((blk, _LANES), -3.0, jnp.float32) for _ in range(nsel)]
    es = jnp.zeros((blk, _LANES), jnp.float32)
    for g in range(B // _LANES):
        v = jax.lax.slice(sim, (0, g * _LANES), (blk, (g + 1) * _LANES))
        es = es + jnp.exp(v * inv_t)
        for j in range(nsel):
            hi = jnp.maximum(tops[j], v)
            v = jnp.minimum(tops[j], v)
            tops[j] = hi
    esum = jnp.sum(es, axis=1, keepdims=True)

    work = jnp.concatenate(tops, axis=1)
    top_sum = jnp.zeros((blk, 1), jnp.float32)
    exp_top = jnp.zeros((blk, 1), jnp.float32)
    taken = jnp.zeros((blk, 1), jnp.float32)
    for k in range(nsel):
        m = jnp.max(work, axis=1, keepdims=True)
        eq = work == m
        cnt = jnp.sum(eq.astype(jnp.float32), axis=1, keepdims=True)
        use = jnp.minimum(cnt, nsel - taken)
        contrib = jnp.maximum(use - jnp.maximum(1.0 - taken, 0.0), 0.0)
        top_sum = top_sum + m * contrib
        exp_top = exp_top + jnp.exp(m * inv_t) * use
        taken = taken + use
        if k + 1 < nsel:
            work = jnp.where(eq, -3.0, work)

    neg_sum = esum - exp_top
    row_loss = top_sum * inv_t - _SIGMA_STATIC * jnp.log(neg_sum)
    partial = jnp.sum(row_loss).reshape(1, 1)

    @pl.when(i == 0)
    def _init():
        acc_ref[...] = jnp.zeros((1, 1), jnp.float32)

    acc_ref[...] += partial


def kernel(features, sigma):
    B, D = features.shape
    blk = min(_ROW_BLOCK, B)
    grid = (B // blk,)

    acc = pl.pallas_call(
        functools.partial(_cts_loss_kernel, blk=blk),
        grid=grid,
        in_specs=[pl.BlockSpec((B, D), lambda i: (0, 0))],
        out_specs=pl.BlockSpec((1, 1), lambda i: (0, 0)),
        out_shape=jax.ShapeDtypeStruct((1, 1), jnp.float32),
        scratch_shapes=[pltpu.VMEM((B, D), jnp.float32)],
    )(features)

    loss = -(acc[0, 0] / sigma) / B
    return jnp.maximum(loss, jnp.asarray(0.0, dtype=loss.dtype))


# speedup vs baseline: 1.7589x; 1.0295x over previous
import functools

import jax
import jax.numpy as jnp
from jax.experimental import pallas as pl
from jax.experimental.pallas import tpu as pltpu

_TEMPERATURE = 0.1
_SIGMA_STATIC = 5
_ROW_BLOCK = 2048
_LANES = 128


def _cts_loss_kernel(f_ref, acc_ref, fn_ref, *, blk):
    i = pl.program_id(0)
    B = f_ref.shape[0]
    nsel = _SIGMA_STATIC + 1
    inv_t = 1.0 / _TEMPERATURE

    @pl.when(i == 0)
    def _normalize():
        f = f_ref[...]
        norm = jnp.sqrt(jnp.sum(f * f, axis=1, keepdims=True))
        fn_ref[...] = f / jnp.maximum(norm, 1e-12)

    fn = fn_ref[...]
    fblk = fn_ref[pl.ds(i * blk, blk), :]

    sim = jax.lax.dot_general(
        fblk, fn,
        dimension_numbers=(((1,), (1,)), ((), ())),
        preferred_element_type=jnp.float32,
    )

    tops = [jnp.full((blk, _LANES), -3.0, jnp.float32) for _ in range(nsel)]
    es = jnp.zeros((blk, _LANES), jnp.float32)
    for g in range(B // _LANES):
        v = jax.lax.slice(sim, (0, g * _LANES), (blk, (g + 1) * _LANES))
        es = es + jnp.exp(v * inv_t)
        for j in range(nsel):
            hi = jnp.maximum(tops[j], v)
            v = jnp.minimum(tops[j], v)
            tops[j] = hi
    esum = jnp.sum(es, axis=1, keepdims=True)

    work = jnp.concatenate(tops, axis=1)
    top_sum = jnp.zeros((blk, 1), jnp.float32)
    exp_top = jnp.zeros((blk, 1), jnp.float32)
    taken = jnp.zeros((blk, 1), jnp.float32)
    for k in range(nsel):
        m = jnp.max(work, axis=1, keepdims=True)
        eq = work == m
        cnt = jnp.sum(eq.astype(jnp.float32), axis=1, keepdims=True)
        use = jnp.minimum(cnt, nsel - taken)
        contrib = jnp.maximum(use - jnp.maximum(1.0 - taken, 0.0), 0.0)
        top_sum = top_sum + m * contrib
        exp_top = exp_top + jnp.exp(m * inv_t) * use
        taken = taken + use
        if k + 1 < nsel:
            work = jnp.where(eq, -3.0, work)

    neg_sum = esum - exp_top
    row_loss = top_sum * inv_t - _SIGMA_STATIC * jnp.log(neg_sum)
    partial = jnp.sum(row_loss).reshape(1, 1)

    @pl.when(i == 0)
    def _init():
        acc_ref[...] = jnp.zeros((1, 1), jnp.float32)

    acc_ref[...] += partial


def kernel(features, sigma):
    B, D = features.shape
    blk = min(_ROW_BLOCK, B)
    grid = (B // blk,)

    acc = pl.pallas_call(
        functools.partial(_cts_loss_kernel, blk=blk),
        grid=grid,
        in_specs=[pl.BlockSpec((B, D), lambda i: (0, 0))],
        out_specs=pl.BlockSpec((1, 1), lambda i: (0, 0)),
        out_shape=jax.ShapeDtypeStruct((1, 1), jnp.float32),
        scratch_shapes=[pltpu.VMEM((B, D), jnp.float32)],
    )(features)

    loss = -(acc[0, 0] / sigma) / B
    return jnp.maximum(loss, jnp.asarray(0.0, dtype=loss.dtype))


# speedup vs baseline: 2.1759x; 1.2371x over previous
import functools

import jax
import jax.numpy as jnp
from jax.experimental import pallas as pl
from jax.experimental.pallas import tpu as pltpu

_TEMPERATURE = 0.1
_SIGMA_STATIC = 5
_ROW_BLOCK = 2048
_LANES = 128


def _cts_loss_kernel(f_ref, acc_ref, fn_ref, *, blk):
    i = pl.program_id(0)
    B = f_ref.shape[0]
    nsel = _SIGMA_STATIC + 1
    inv_t = 1.0 / _TEMPERATURE

    @pl.when(i == 0)
    def _normalize():
        f = f_ref[...]
        norm = jnp.sqrt(jnp.sum(f * f, axis=1, keepdims=True))
        fn_ref[...] = f / jnp.maximum(norm, 1e-12)

    fn = fn_ref[...]
    fblk = fn_ref[pl.ds(i * blk, blk), :]

    sim = jax.lax.dot_general(
        fblk, fn,
        dimension_numbers=(((1,), (1,)), ((), ())),
        preferred_element_type=jnp.float32,
    )

    inv_t_log2e = inv_t * 1.4426950408889634
    es = jnp.zeros((blk, _LANES), jnp.float32)
    slabs = []
    for g in range(B // _LANES):
        v = jax.lax.slice(sim, (0, g * _LANES), (blk, (g + 1) * _LANES))
        es = es + jax.lax.exp2(v * inv_t_log2e)
        slabs.append(v)
    esum = jnp.sum(es, axis=1, keepdims=True)

    def _s2(a, b):
        return jnp.maximum(a, b), jnp.minimum(a, b)

    def _merge22(a, b):
        z1, d1 = _s2(a[0], b[0])
        c2, z4 = _s2(a[1], b[1])
        z2, z3 = _s2(d1, c2)
        return (z1, z2, z3, z4)

    def _merge44(a, b):
        o = _merge22((a[0], a[2]), (b[0], b[2]))
        e = _merge22((a[1], a[3]), (b[1], b[3]))
        z2, z3 = _s2(e[0], o[1])
        z4, z5 = _s2(e[1], o[2])
        z6, _ = _s2(e[2], o[3])
        return (o[0], z2, z3, z4, z5, z6)

    def _sort3(a, b, c):
        a, b = _s2(a, b)
        a, c = _s2(a, c)
        b, c = _s2(b, c)
        return a, b, c

    def _merge66_top6(a, b, sort_output):
        m = [jnp.maximum(a[i], b[5 - i]) for i in range(6)]
        if not sort_output:
            return tuple(m)
        hi = [jnp.maximum(m[i], m[i + 3]) for i in range(3)]
        lo = [jnp.minimum(m[i], m[i + 3]) for i in range(3)]
        return (*_sort3(*hi), *_sort3(*lo))

    s2 = [_s2(slabs[2 * i], slabs[2 * i + 1]) for i in range(16)]
    s4 = [_merge22(s2[2 * i], s2[2 * i + 1]) for i in range(8)]
    s6 = [_merge44(s4[2 * i], s4[2 * i + 1]) for i in range(4)]
    s6b = [_merge66_top6(s6[2 * i], s6[2 * i + 1], True) for i in range(2)]
    tops = _merge66_top6(s6b[0], s6b[1], False)

    work = jnp.concatenate(tops, axis=1)
    top_sum = jnp.zeros((blk, 1), jnp.float32)
    exp_top = jnp.zeros((blk, 1), jnp.float32)
    taken = jnp.zeros((blk, 1), jnp.float32)
    for k in range(nsel):
        m = jnp.max(work, axis=1, keepdims=True)
        eq = work == m
        cnt = jnp.sum(eq.astype(jnp.float32), axis=1, keepdims=True)
        use = jnp.minimum(cnt, nsel - taken)
        contrib = jnp.maximum(use - jnp.maximum(1.0 - taken, 0.0), 0.0)
        top_sum = top_sum + m * contrib
        exp_top = exp_top + jax.lax.exp2(m * inv_t_log2e) * use
        taken = taken + use
        if k + 1 < nsel:
            work = jnp.where(eq, -3.0, work)

    neg_sum = esum - exp_top
    row_loss = top_sum * inv_t - _SIGMA_STATIC * jnp.log(neg_sum)
    partial = jnp.sum(row_loss).reshape(1, 1)

    @pl.when(i == 0)
    def _init():
        acc_ref[...] = jnp.zeros((1, 1), jnp.float32)

    acc_ref[...] += partial


def kernel(features, sigma):
    B, D = features.shape
    blk = min(_ROW_BLOCK, B)
    grid = (B // blk,)

    acc = pl.pallas_call(
        functools.partial(_cts_loss_kernel, blk=blk),
        grid=grid,
        in_specs=[pl.BlockSpec((B, D), lambda i: (0, 0))],
        out_specs=pl.BlockSpec((1, 1), lambda i: (0, 0)),
        out_shape=jax.ShapeDtypeStruct((1, 1), jnp.float32),
        scratch_shapes=[pltpu.VMEM((B, D), jnp.float32)],
    )(features)

    loss = -(acc[0, 0] / sigma) / B
    return jnp.maximum(loss, jnp.asarray(0.0, dtype=loss.dtype))


# speedup vs baseline: 2.4729x; 1.1365x over previous
import functools

import jax
import jax.numpy as jnp
from jax.experimental import pallas as pl
from jax.experimental.pallas import tpu as pltpu

_TEMPERATURE = 0.1
_SIGMA_STATIC = 5
_ROW_BLOCK = 2048
_LANES = 128


def _cts_loss_kernel(f_ref, acc_ref, fn_ref, *, blk):
    i = pl.program_id(0)
    B = f_ref.shape[0]
    nsel = _SIGMA_STATIC + 1
    inv_t = 1.0 / _TEMPERATURE

    @pl.when(i == 0)
    def _normalize():
        f = f_ref[...]
        norm = jnp.sqrt(jnp.sum(f * f, axis=1, keepdims=True))
        fn_ref[...] = f / jnp.maximum(norm, 1e-12)

    fn = fn_ref[...]
    fblk = fn_ref[pl.ds(i * blk, blk), :]

    sim = jax.lax.dot_general(
        fblk, fn,
        dimension_numbers=(((1,), (1,)), ((), ())),
        preferred_element_type=jnp.float32,
    )

    inv_t_log2e = inv_t * 1.4426950408889634
    es = jnp.zeros((blk, _LANES), jnp.float32)
    slabs = []
    for g in range(B // _LANES):
        v = jax.lax.slice(sim, (0, g * _LANES), (blk, (g + 1) * _LANES))
        es = es + jax.lax.exp2(v * inv_t_log2e)
        slabs.append(v)
    esum = jnp.sum(es, axis=1, keepdims=True)

    def _s2(a, b):
        return jnp.maximum(a, b), jnp.minimum(a, b)

    def _merge22(a, b):
        z1, d1 = _s2(a[0], b[0])
        c2, z4 = _s2(a[1], b[1])
        z2, z3 = _s2(d1, c2)
        return (z1, z2, z3, z4)

    def _merge44(a, b):
        o = _merge22((a[0], a[2]), (b[0], b[2]))
        e = _merge22((a[1], a[3]), (b[1], b[3]))
        z2, z3 = _s2(e[0], o[1])
        z4, z5 = _s2(e[1], o[2])
        z6, _ = _s2(e[2], o[3])
        return (o[0], z2, z3, z4, z5, z6)

    def _sort3(a, b, c):
        a, b = _s2(a, b)
        a, c = _s2(a, c)
        b, c = _s2(b, c)
        return a, b, c

    def _merge66_top6(a, b, sort_output):
        m = [jnp.maximum(a[i], b[5 - i]) for i in range(6)]
        if not sort_output:
            return tuple(m)
        hi = [jnp.maximum(m[i], m[i + 3]) for i in range(3)]
        lo = [jnp.minimum(m[i], m[i + 3]) for i in range(3)]
        return (*_sort3(*hi), *_sort3(*lo))

    s2 = [_s2(slabs[2 * i], slabs[2 * i + 1]) for i in range(16)]
    s4 = [_merge22(s2[2 * i], s2[2 * i + 1]) for i in range(8)]
    s6 = [_merge44(s4[2 * i], s4[2 * i + 1]) for i in range(4)]
    s6b = [_merge66_top6(s6[2 * i], s6[2 * i + 1], True) for i in range(2)]
    t = list(_merge66_top6(s6b[0], s6b[1], True))

    top_sum = jnp.zeros((blk, 1), jnp.float32)
    exp_top = jnp.zeros((blk, 1), jnp.float32)
    taken = jnp.zeros((blk, 1), jnp.float32)
    for k in range(nsel):
        m = jnp.max(t[0], axis=1, keepdims=True)
        eq = t[0] == m
        cnt = jnp.sum(eq.astype(jnp.float32), axis=1, keepdims=True)
        use = jnp.minimum(cnt, nsel - taken)
        contrib = jnp.maximum(use - jnp.maximum(1.0 - taken, 0.0), 0.0)
        top_sum = top_sum + m * contrib
        exp_top = exp_top + jax.lax.exp2(m * inv_t_log2e) * use
        taken = taken + use
        if k + 1 < nsel:
            for j in range(nsel - 1):
                t[j] = jnp.where(eq, t[j + 1], t[j])
            t[nsel - 1] = jnp.where(eq, -3.0, t[nsel - 1])

    neg_sum = esum - exp_top
    row_loss = top_sum * inv_t - _SIGMA_STATIC * jnp.log(neg_sum)
    partial = jnp.sum(row_loss).reshape(1, 1)

    @pl.when(i == 0)
    def _init():
        acc_ref[...] = jnp.zeros((1, 1), jnp.float32)

    acc_ref[...] += partial


def kernel(features, sigma):
    B, D = features.shape
    blk = min(_ROW_BLOCK, B)
    grid = (B // blk,)

    acc = pl.pallas_call(
        functools.partial(_cts_loss_kernel, blk=blk),
        grid=grid,
        in_specs=[pl.BlockSpec((B, D), lambda i: (0, 0))],
        out_specs=pl.BlockSpec((1, 1), lambda i: (0, 0)),
        out_shape=jax.ShapeDtypeStruct((1, 1), jnp.float32),
        scratch_shapes=[pltpu.VMEM((B, D), jnp.float32)],
    )(features)

    loss = -(acc[0, 0] / sigma) / B
    return jnp.maximum(loss, jnp.asarray(0.0, dtype=loss.dtype))
